# baseline (speedup 1.0000x reference)
import functools
import math

import jax
import jax.numpy as jnp
from jax import lax
from jax.experimental import pallas as pl
from jax.experimental.pallas import tpu as pltpu
from jax.experimental.pallas import tpu_sc as plsc

D_MODEL = 64
LANES = 16
NUM_CORES = 2
NUM_SUBCORES = 16
NUM_WORKERS = NUM_CORES * NUM_SUBCORES
QSZ = (64, 48, 48, 40)
QOFF = (0, 64, 112, 160)
CHUNK_MAX = max(QSZ)
NBUF = len(QSZ)


def _emb_body(rows_per_w, seq, scale, x_hbm, table_hbm, out_hbm, idx_v, idxg_v,
              raw_v, scl_v, gsem, osem):
  cid = lax.axis_index("c")
  sid = lax.axis_index("s")
  wid = sid * NUM_CORES + cid
  row0 = wid * rows_per_w

  pltpu.sync_copy(x_hbm.at[pl.ds(row0, rows_per_w)], idx_v)

  def pair_ids(bl, q, b):
    sz, off = QSZ[q], QOFF[q]
    for k in range((sz + LANES - 1) // LANES):
      k0 = min(k * LANES, sz - LANES)
      idxg_v[b, pl.ds(k0, LANES)] = idx_v[bl, pl.ds(off + k0, LANES)] >> 1

  def gather_start(b, q):
    sz = QSZ[q]
    pltpu.async_copy(table_hbm.at[idxg_v.at[b, pl.ds(0, sz)]],
                     raw_v.at[b, pl.ds(0, sz)], gsem.at[b])

  def gather_wait(b, q):
    sz = QSZ[q]
    pltpu.make_async_copy(table_hbm.at[idxg_v.at[0, pl.ds(0, sz)]],
                          raw_v.at[b, pl.ds(0, sz)], gsem.at[b]).wait()

  def out_start(bl, q, b):
    sz, off = QSZ[q], QOFF[q]
    pltpu.async_copy(scl_v.at[b, pl.ds(0, sz)],
                     out_hbm.at[row0 + bl, pl.ds(off, sz)], osem.at[b])

  def out_wait(q, b):
    sz = QSZ[q]
    pltpu.make_async_copy(scl_v.at[b, pl.ds(0, sz)],
                          out_hbm.at[0, pl.ds(0, sz)], osem.at[b]).wait()

  def select_scale(bl, q, b):
    sz, off = QSZ[q], QOFF[q]
    blocks = [(k * LANES, 0) for k in range(sz // LANES)]
    if sz % LANES:
      blocks.append((sz - LANES, LANES - sz % LANES))
    for rbase, r0 in blocks:
      par = (idx_v[bl, pl.ds(off + rbase, LANES)] & 1) * D_MODEL
      for r in range(r0, LANES):
        base = par[r]
        rr = rbase + r
        for j in range(D_MODEL // LANES):
          scl_v[b, rr, pl.ds(j * LANES, LANES)] = (
              raw_v[b, rr, pl.ds(base + j * LANES, LANES)] * scale)

  for b in range(NBUF):
    pair_ids(jnp.int32(0), b, b)
    gather_start(b, b)

  def group(bl, carry):
    for b in range(NBUF):
      q = b
      gather_wait(b, q)

      @pl.when(bl > 0)
      def _():
        out_wait(q, b)

      select_scale(bl, q, b)
      out_start(bl, q, b)

      @pl.when(bl + 1 < rows_per_w)
      def _():
        pair_ids(bl + 1, q, b)
        gather_start(b, q)

    return carry

  lax.fori_loop(0, rows_per_w, group, 0)

  for b in range(NBUF):
    out_wait(b, b)


def kernel(x, table):
  bsz, seq = x.shape
  vocab, d = table.shape
  assert d == D_MODEL
  assert seq == sum(QSZ)
  assert bsz % NUM_WORKERS == 0
  rows_per_w = bsz // NUM_WORKERS

  table2 = table.reshape(vocab // 2, 2 * d)
  scale = jnp.float32(math.sqrt(d))

  mesh = plsc.VectorSubcoreMesh(
      core_axis_name="c", subcore_axis_name="s",
      num_cores=NUM_CORES, num_subcores=NUM_SUBCORES)

  emb = pl.kernel(
      functools.partial(_emb_body, rows_per_w, seq, scale),
      out_type=jax.ShapeDtypeStruct((bsz, seq, d), jnp.float32),
      mesh=mesh,
      compiler_params=pltpu.CompilerParams(
          use_tc_tiling_on_sc=True, needs_layout_passes=False),
      scratch_types=[
          pltpu.VMEM((rows_per_w, seq), jnp.int32),
          pltpu.VMEM((NBUF, CHUNK_MAX), jnp.int32),
          pltpu.VMEM((NBUF, CHUNK_MAX, 2 * d), jnp.float32),
          pltpu.VMEM((NBUF, CHUNK_MAX, d), jnp.float32),
          pltpu.SemaphoreType.DMA((NBUF,)),
          pltpu.SemaphoreType.DMA((NBUF,)),
      ],
  )(x.astype(jnp.int32), table2)

  return emb


# speedup vs baseline: 1.6554x; 1.6554x over previous
import functools
import math

import jax
import jax.numpy as jnp
from jax import lax
from jax.experimental import pallas as pl
from jax.experimental.pallas import tpu as pltpu
from jax.experimental.pallas import tpu_sc as plsc

D_MODEL = 64
LANES = 16
NUM_CORES = 2
NUM_SUBCORES = 16
NUM_WORKERS = NUM_CORES * NUM_SUBCORES
SPLIT = (104, 96)
CHUNK_MAX = max(SPLIT)
NBUF = 4


def _emb_body(rows_per_w, seq, scale, x_hbm, table_hbm, out_hbm, idx_v, raw_v,
              scl_v, gsem, osem):
  cid = lax.axis_index("c")
  sid = lax.axis_index("s")
  wid = sid * NUM_CORES + cid
  row0 = wid * rows_per_w

  pltpu.sync_copy(x_hbm.at[pl.ds(row0, rows_per_w)], idx_v)

  def gather_start(bl, half, b):
    sz = SPLIT[half]
    s0 = SPLIT[0] * half
    pltpu.async_copy(table_hbm.at[idx_v.at[bl, pl.ds(s0, sz)]],
                     raw_v.at[b, pl.ds(0, sz)], gsem.at[b])

  def gather_wait(half, b):
    sz = SPLIT[half]
    pltpu.make_async_copy(table_hbm.at[idx_v.at[0, pl.ds(0, sz)]],
                          raw_v.at[b, pl.ds(0, sz)], gsem.at[b]).wait()

  def out_start(bl, half, b):
    sz = SPLIT[half]
    s0 = SPLIT[0] * half
    base = (row0 + bl) * seq + s0
    pltpu.async_copy(scl_v.at[b, pl.ds(0, sz)],
                     out_hbm.at[pl.ds(base, sz), pl.ds(0, D_MODEL)],
                     osem.at[b])

  def out_wait(half, b):
    sz = SPLIT[half]
    pltpu.make_async_copy(scl_v.at[b, pl.ds(0, sz)],
                          out_hbm.at[pl.ds(0, sz), pl.ds(0, D_MODEL)],
                          osem.at[b]).wait()

  half_of = [b % 2 for b in range(NBUF)]
  bl_of = [b // 2 for b in range(NBUF)]

  for b in range(NBUF):
    gather_start(jnp.int32(bl_of[b]), half_of[b], b)

  rows_per_group = NBUF // 2

  def group(g, carry):
    for b in range(NBUF):
      half = half_of[b]
      sz = SPLIT[half]
      bl = g * rows_per_group + bl_of[b]
      gather_wait(half, b)

      @pl.when(g > 0)
      def _():
        out_wait(half, b)

      @plsc.parallel_loop(0, sz, unroll=8)
      def _(r):
        for j in range(D_MODEL // LANES):
          sl = pl.ds(j * LANES, LANES)
          scl_v[b, r, sl] = raw_v[b, r, sl] * scale

      out_start(bl, half, b)

      @pl.when(bl + rows_per_group < rows_per_w)
      def _():
        gather_start(bl + rows_per_group, half, b)

    return carry

  lax.fori_loop(0, rows_per_w // rows_per_group, group, 0)

  for b in range(NBUF):
    out_wait(half_of[b], b)


def kernel(x, table):
  bsz, seq = x.shape
  vocab, d = table.shape
  assert d == D_MODEL
  assert seq == sum(SPLIT)
  assert bsz % NUM_WORKERS == 0
  rows_per_w = bsz // NUM_WORKERS
  assert rows_per_w % (NBUF // 2) == 0

  scale = jnp.float32(math.sqrt(d))

  mesh = plsc.VectorSubcoreMesh(
      core_axis_name="c", subcore_axis_name="s",
      num_cores=NUM_CORES, num_subcores=NUM_SUBCORES)

  o2 = pl.kernel(
      functools.partial(_emb_body, rows_per_w, seq, scale),
      out_type=jax.ShapeDtypeStruct((bsz * seq, 2 * d), jnp.float32),
      mesh=mesh,
      compiler_params=pltpu.CompilerParams(use_tc_tiling_on_sc=False),
      scratch_types=[
          pltpu.VMEM((rows_per_w, seq), jnp.int32),
          pltpu.VMEM((NBUF, CHUNK_MAX, d), jnp.float32),
          pltpu.VMEM((NBUF, CHUNK_MAX, d), jnp.float32),
          pltpu.SemaphoreType.DMA((NBUF,)),
          pltpu.SemaphoreType.DMA((NBUF,)),
      ],
  )(x.astype(jnp.int32), table)

  return o2[:, :d].reshape(bsz, seq, d)
